# baseline (speedup 1.0000x reference)
import functools

import jax
import jax.numpy as jnp
from jax.experimental import pallas as pl
from jax.experimental.pallas import tpu as pltpu

D_MODEL = 768
D_FF = 2048
N_EXPERTS = 8
AUX_COEF = 0.01
TM = 256


def _quant_body(w_ref, q_ref):
    w = w_ref[0]
    t = 0.5 * jnp.mean(jnp.abs(w))
    q_ref[0] = ((w > t).astype(jnp.float32) - (w < -t).astype(jnp.float32)
                ).astype(jnp.bfloat16)


def _quantize(w):
    e, a, b = w.shape
    return pl.pallas_call(
        _quant_body,
        grid=(e,),
        in_specs=[pl.BlockSpec((1, a, b), lambda i: (i, 0, 0))],
        out_specs=pl.BlockSpec((1, a, b), lambda i: (i, 0, 0)),
        out_shape=jax.ShapeDtypeStruct((e, a, b), jnp.bfloat16),
    )(w)


def _router_body(x_ref, rw_ref, combine_ref, aux_ref):
    x = x_ref[...]
    rw = rw_ref[...]
    dn = (((1,), (1,)), ((), ()))
    logits = jax.lax.dot_general(x, rw, dn, preferred_element_type=jnp.float32)
    p = jax.nn.softmax(logits, axis=-1)
    n, e = p.shape
    lane = jax.lax.broadcasted_iota(jnp.int32, (n, e), 1)
    m1 = jnp.max(p, axis=1, keepdims=True)
    i1 = jnp.min(jnp.where(p >= m1, lane, e), axis=1, keepdims=True)
    mask1 = lane == i1
    p2 = jnp.where(mask1, -1.0, p)
    m2 = jnp.max(p2, axis=1, keepdims=True)
    i2 = jnp.min(jnp.where(p2 >= m2, lane, e), axis=1, keepdims=True)
    mask2 = lane == i2
    denom = m1 + m2 + 1e-9
    combine_ref[...] = (m1 / denom) * mask1 + (m2 / denom) * mask2
    mp = jnp.mean(p, axis=0)
    entropy = -jnp.sum(mp * jnp.log(mp + 1e-9))
    aux_ref[...] = jnp.full((1, 1), -entropy * AUX_COEF, jnp.float32)


def _router(xf, router_w):
    n, d = xf.shape
    e = router_w.shape[0]
    return pl.pallas_call(
        _router_body,
        in_specs=[pl.BlockSpec((n, d), lambda: (0, 0)),
                  pl.BlockSpec((e, d), lambda: (0, 0))],
        out_specs=[pl.BlockSpec((n, e), lambda: (0, 0)),
                   pl.BlockSpec((1, 1), lambda: (0, 0))],
        out_shape=[jax.ShapeDtypeStruct((n, e), jnp.float32),
                   jax.ShapeDtypeStruct((1, 1), jnp.float32)],
    )(xf, router_w)


def _expert_body(xb_ref, combine_ref, wg_ref, wu_ref, wd_ref,
                 gg_ref, ug_ref, dg_ref, out_ref):
    e = pl.program_id(0)
    m = pl.program_id(1)
    x = xb_ref[...]
    dn = (((1,), (1,)), ((), ()))
    g = jax.lax.dot_general(x, wg_ref[0], dn,
                            preferred_element_type=jnp.float32) * gg_ref[e]
    u = jax.lax.dot_general(x, wu_ref[0], dn,
                            preferred_element_type=jnp.float32) * ug_ref[e]
    h = (g * jax.nn.sigmoid(g) * u).astype(jnp.bfloat16)
    y = jax.lax.dot_general(h, wd_ref[0], dn,
                            preferred_element_type=jnp.float32) * dg_ref[e]
    cm = combine_ref[...]
    lane = jax.lax.broadcasted_iota(jnp.int32, cm.shape, 1)
    c = jnp.sum(jnp.where(lane == e, cm, 0.0), axis=1)
    yw = y * c[:, None]
    row = pl.ds(m * TM, TM)

    @pl.when(e == 0)
    def _():
        out_ref[row, :] = yw

    @pl.when(e > 0)
    def _():
        out_ref[row, :] += yw


def _experts(xb, combine, wg_q, wu_q, wd_q, gg, ug, dg):
    n, d = xb.shape
    ne = wg_q.shape[0]
    ff = wg_q.shape[1]
    grid = (ne, n // TM)
    return pl.pallas_call(
        _expert_body,
        grid=grid,
        in_specs=[
            pl.BlockSpec((TM, d), lambda e, m: (m, 0)),
            pl.BlockSpec((TM, ne), lambda e, m: (m, 0)),
            pl.BlockSpec((1, ff, d), lambda e, m: (e, 0, 0)),
            pl.BlockSpec((1, ff, d), lambda e, m: (e, 0, 0)),
            pl.BlockSpec((1, d, ff), lambda e, m: (e, 0, 0)),
            pl.BlockSpec(memory_space=pltpu.SMEM),
            pl.BlockSpec(memory_space=pltpu.SMEM),
            pl.BlockSpec(memory_space=pltpu.SMEM),
        ],
        out_specs=pl.BlockSpec((n, d), lambda e, m: (0, 0)),
        out_shape=jax.ShapeDtypeStruct((n, d), jnp.float32),
        compiler_params=pltpu.CompilerParams(
            dimension_semantics=("arbitrary", "arbitrary")),
    )(xb, combine, wg_q, wu_q, wd_q, gg, ug, dg)


def kernel(x, router_w, gate_w, up_w, down_w, gate_gamma, up_gamma, down_gamma):
    b, t, d = x.shape
    xf = x.reshape(-1, d)
    wg_q = _quantize(gate_w)
    wu_q = _quantize(up_w)
    wd_q = _quantize(down_w)
    combine, aux = _router(xf, router_w)
    xb = xf.astype(jnp.bfloat16)
    out = _experts(xb, combine, wg_q, wu_q, wd_q,
                   gate_gamma, up_gamma, down_gamma)
    return out.reshape(b, t, d), aux[0, 0]


# speedup vs baseline: 1.0828x; 1.0828x over previous
import functools

import jax
import jax.numpy as jnp
from jax.experimental import pallas as pl
from jax.experimental.pallas import tpu as pltpu

D_MODEL = 768
D_FF = 2048
N_EXPERTS = 8
AUX_COEF = 0.01
TM = 256


def _quant_body(w_ref, q_ref):
    w = w_ref[0]
    t = 0.5 * jnp.mean(jnp.abs(w))
    q_ref[0] = ((w > t).astype(jnp.float32) - (w < -t).astype(jnp.float32)
                ).astype(jnp.bfloat16)


def _quantize(w):
    e, a, b = w.shape
    return pl.pallas_call(
        _quant_body,
        grid=(e,),
        in_specs=[pl.BlockSpec((1, a, b), lambda i: (i, 0, 0))],
        out_specs=pl.BlockSpec((1, a, b), lambda i: (i, 0, 0)),
        out_shape=jax.ShapeDtypeStruct((e, a, b), jnp.bfloat16),
    )(w)


def _router_body(x_ref, rw_ref, combine_ref, aux_ref):
    x = x_ref[...]
    rw = rw_ref[...]
    dn = (((1,), (1,)), ((), ()))
    logits = jax.lax.dot_general(x, rw, dn, preferred_element_type=jnp.float32)
    p = jax.nn.softmax(logits, axis=-1)
    n, e = p.shape
    lane = jax.lax.broadcasted_iota(jnp.int32, (n, e), 1)
    m1 = jnp.max(p, axis=1, keepdims=True)
    i1 = jnp.min(jnp.where(p >= m1, lane, e), axis=1, keepdims=True)
    mask1 = lane == i1
    p2 = jnp.where(mask1, -1.0, p)
    m2 = jnp.max(p2, axis=1, keepdims=True)
    i2 = jnp.min(jnp.where(p2 >= m2, lane, e), axis=1, keepdims=True)
    mask2 = lane == i2
    denom = m1 + m2 + 1e-9
    combine_ref[...] = (m1 / denom) * mask1 + (m2 / denom) * mask2
    mp = jnp.mean(p, axis=0)
    entropy = -jnp.sum(mp * jnp.log(mp + 1e-9))
    aux_ref[...] = jnp.full((1, 1), -entropy * AUX_COEF, jnp.float32)


def _router(xf, router_w):
    n, d = xf.shape
    e = router_w.shape[0]
    return pl.pallas_call(
        _router_body,
        in_specs=[pl.BlockSpec((n, d), lambda: (0, 0)),
                  pl.BlockSpec((e, d), lambda: (0, 0))],
        out_specs=[pl.BlockSpec((n, e), lambda: (0, 0)),
                   pl.BlockSpec((1, 1), lambda: (0, 0))],
        out_shape=[jax.ShapeDtypeStruct((n, e), jnp.float32),
                   jax.ShapeDtypeStruct((1, 1), jnp.float32)],
    )(xf, router_w)


def _expert_body(xb_ref, combine_ref, wg_ref, wu_ref, wd_ref,
                 gg_ref, ug_ref, dg_ref, out_ref,
                 wgq_s, wuq_s, wdq_s):
    e = pl.program_id(0)
    m = pl.program_id(1)

    @pl.when(m == 0)
    def _():
        for src, dst in ((wg_ref, wgq_s), (wu_ref, wuq_s), (wd_ref, wdq_s)):
            a, b = dst.shape
            nc = 8
            ch = a // nc
            s = jnp.float32(0.0)
            for i in range(nc):
                s += jnp.sum(jnp.abs(src[0, i * ch:(i + 1) * ch, :]))
            t = 0.5 * s / (a * b)
            for i in range(nc):
                w = src[0, i * ch:(i + 1) * ch, :]
                dst[i * ch:(i + 1) * ch, :] = (
                    (w > t).astype(jnp.float32)
                    - (w < -t).astype(jnp.float32)).astype(jnp.bfloat16)

    x = xb_ref[...]
    dn = (((1,), (1,)), ((), ()))
    g = jax.lax.dot_general(x, wgq_s[...], dn,
                            preferred_element_type=jnp.float32) * gg_ref[e]
    u = jax.lax.dot_general(x, wuq_s[...], dn,
                            preferred_element_type=jnp.float32) * ug_ref[e]
    h = (g * jax.nn.sigmoid(g) * u).astype(jnp.bfloat16)
    y = jax.lax.dot_general(h, wdq_s[...], dn,
                            preferred_element_type=jnp.float32) * dg_ref[e]
    cm = combine_ref[...]
    lane = jax.lax.broadcasted_iota(jnp.int32, cm.shape, 1)
    c = jnp.sum(jnp.where(lane == e, cm, 0.0), axis=1)
    yw = y * c[:, None]
    row = pl.ds(m * TM, TM)

    @pl.when(e == 0)
    def _():
        out_ref[row, :] = yw

    @pl.when(e > 0)
    def _():
        out_ref[row, :] += yw


def _experts(xb, combine, wg_q, wu_q, wd_q, gg, ug, dg):
    n, d = xb.shape
    ne = wg_q.shape[0]
    ff = wg_q.shape[1]
    grid = (ne, n // TM)
    return pl.pallas_call(
        _expert_body,
        grid=grid,
        in_specs=[
            pl.BlockSpec((TM, d), lambda e, m: (m, 0)),
            pl.BlockSpec((TM, ne), lambda e, m: (m, 0)),
            pl.BlockSpec((1, ff, d), lambda e, m: (e, 0, 0)),
            pl.BlockSpec((1, ff, d), lambda e, m: (e, 0, 0)),
            pl.BlockSpec((1, d, ff), lambda e, m: (e, 0, 0)),
            pl.BlockSpec(memory_space=pltpu.SMEM),
            pl.BlockSpec(memory_space=pltpu.SMEM),
            pl.BlockSpec(memory_space=pltpu.SMEM),
        ],
        out_specs=pl.BlockSpec((n, d), lambda e, m: (0, 0)),
        out_shape=jax.ShapeDtypeStruct((n, d), jnp.float32),
        scratch_shapes=[pltpu.VMEM((ff, d), jnp.bfloat16),
                        pltpu.VMEM((ff, d), jnp.bfloat16),
                        pltpu.VMEM((d, ff), jnp.bfloat16)],
        compiler_params=pltpu.CompilerParams(
            dimension_semantics=("arbitrary", "arbitrary"),
            vmem_limit_bytes=100 * 1024 * 1024),
    )(xb, combine, wg_q, wu_q, wd_q, gg, ug, dg)


def kernel(x, router_w, gate_w, up_w, down_w, gate_gamma, up_gamma, down_gamma):
    b, t, d = x.shape
    xf = x.reshape(-1, d)
    combine, aux = _router(xf, router_w)
    xb = xf.astype(jnp.bfloat16)
    out = _experts(xb, combine, gate_w, up_w, down_w,
                   gate_gamma, up_gamma, down_gamma)
    return out.reshape(b, t, d), aux[0, 0]
